# baseline (speedup 1.0000x reference)
import functools

import jax
import jax.numpy as jnp
import numpy as np
from jax import lax
from jax.experimental import pallas as pl
from jax.experimental.pallas import tpu as pltpu
from jax.experimental.pallas import tpu_sc as plsc

_BATCH = 16384
_DIM = 64
_GAMMA = 12.0
_NW = 32
_BPW = _BATCH // _NW
_CHUNK = 64
_NCHUNK = _BPW // _CHUNK
_GROUPS = _CHUNK // 16
_BL = 2048


def _pack_body(eye_ref, lo_ref, hi_ref, out_ref):
    out_ref[:, 0:_DIM] = jax.lax.dot_general(
        lo_ref[...], eye_ref[...],
        dimension_numbers=(((0,), (0,)), ((), ())),
        preferred_element_type=jnp.float32,
        precision=jax.lax.Precision.DEFAULT)
    out_ref[:, _DIM:2 * _DIM] = jax.lax.dot_general(
        hi_ref[...], eye_ref[...],
        dimension_numbers=(((0,), (0,)), ((), ())),
        preferred_element_type=jnp.float32,
        precision=jax.lax.Precision.DEFAULT)


def _pack_table(table_t, half_rows):
    n = table_t.shape[1]
    nlo = half_rows // _BL
    max_blk = (n + _BL - 1) // _BL - 1

    def lo_map(i):
        return (0, i)

    def hi_map(i):
        return (0, jnp.minimum(nlo + i, max_blk))

    eye = np.eye(_DIM, dtype=np.float32)
    return pl.pallas_call(
        _pack_body,
        grid=(nlo,),
        in_specs=[
            pl.BlockSpec((_DIM, _DIM), lambda i: (0, 0)),
            pl.BlockSpec((_DIM, _BL), lo_map),
            pl.BlockSpec((_DIM, _BL), hi_map),
        ],
        out_specs=pl.BlockSpec((_BL, 2 * _DIM), lambda i: (i, 0)),
        out_shape=jax.ShapeDtypeStruct((half_rows, 2 * _DIM), jnp.float32),
    )(jnp.asarray(eye), table_t, table_t)


def _tec_body(ent_half, rel_half,
              heads_h, rels_h, tails_h, nheads_h, ntails_h, ent_h, rel_h,
              out_h,
              h_idx, r_idx, t_idx, nh_idx, nt_idx, row_idx,
              h_rows, r_rows, t_rows, nh_rows, nt_rows,
              out_stage, sems):
    wid = lax.axis_index("s") * 2 + lax.axis_index("c")
    base = wid * _BPW

    pltpu.sync_copy(heads_h.at[pl.ds(base, _BPW)], h_idx)
    pltpu.sync_copy(rels_h.at[pl.ds(base, _BPW)], r_idx)
    pltpu.sync_copy(tails_h.at[pl.ds(base, _BPW)], t_idx)
    pltpu.sync_copy(nheads_h.at[pl.ds(base, _BPW)], nh_idx)
    pltpu.sync_copy(ntails_h.at[pl.ds(base, _BPW)], nt_idx)

    idx_bufs = (h_idx, r_idx, t_idx, nh_idx, nt_idx)
    halves = (ent_half, rel_half, ent_half, ent_half, ent_half)

    def shift_body(i, _):
        for b in range(5):
            v = idx_bufs[b][pl.ds(i * 16, 16)]
            hv = jnp.full((16,), halves[b], jnp.int32)
            row_idx[b, pl.ds(i * 16, 16)] = jnp.where(v < hv, v, v - hv)
        return 0

    lax.fori_loop(0, _BPW // 16, shift_body, 0, unroll=2)

    def start_chunk(c, buf_par):
        off = c * _CHUNK
        sem = sems.at[buf_par]
        srcs = (ent_h, rel_h, ent_h, ent_h, ent_h)
        dsts = (h_rows, r_rows, t_rows, nh_rows, nt_rows)
        return [
            pltpu.async_copy(
                srcs[b].at[row_idx.at[b, pl.ds(off, _CHUNK)]],
                dsts[b].at[buf_par], sem)
            for b in range(5)
        ]

    zero = jnp.zeros((16,), jnp.float32)
    v_loss, v_pos, v_neg = zero, zero, zero
    gamma = zero + _GAMMA
    iota = lax.iota(jnp.int32, 16)

    descs = [None, None]
    descs[0] = start_chunk(0, 0)

    for c in range(_NCHUNK):
        buf_par = c % 2
        if c + 1 < _NCHUNK:
            descs[(c + 1) % 2] = start_chunk(c + 1, (c + 1) % 2)
        for d in descs[buf_par]:
            d.wait()

        hb, rb, tb = h_rows.at[buf_par], r_rows.at[buf_par], t_rows.at[buf_par]
        nhb, ntb = nh_rows.at[buf_par], nt_rows.at[buf_par]

        for g in range(_GROUPS):
            goff = c * _CHUNK + g * 16
            rows = iota + g * 16
            cols = []
            for b in range(5):
                v = idx_bufs[b][pl.ds(goff, 16)]
                hv = jnp.full((16,), halves[b], jnp.int32)
                cols.append(jnp.where(v < hv, 0, _DIM))

            def dim_body(d, acc, hb=hb, rb=rb, tb=tb, nhb=nhb, ntb=ntb,
                         rows=rows, cols=cols):
                ap, an = acc
                h = plsc.load_gather(hb, [rows, cols[0] + d])
                r = plsc.load_gather(rb, [rows, cols[1] + d])
                t = plsc.load_gather(tb, [rows, cols[2] + d])
                nh = plsc.load_gather(nhb, [rows, cols[3] + d])
                nt = plsc.load_gather(ntb, [rows, cols[4] + d])
                ap = ap + jnp.abs(h + r - t)
                an = an + jnp.abs(nh + r - nt)
                return ap, an

            ap, an = lax.fori_loop(0, _DIM, dim_body, (zero, zero), unroll=2)
            v_loss = v_loss + jnp.maximum(gamma + ap - an, 0.0)
            v_pos = v_pos + ap
            v_neg = v_neg + an

    out_stage[0, pl.ds(0, 16)] = v_loss
    out_stage[1, pl.ds(0, 16)] = v_pos
    out_stage[2, pl.ds(0, 16)] = v_neg
    pltpu.sync_copy(out_stage, out_h.at[wid])


@jax.jit
def _transe_sc(heads, relations, tails, negative_heads, negative_tails,
               entity_emb, relation_emb):
    n_ent = entity_emb.shape[0]
    n_rel = relation_emb.shape[0]
    ent_half = 1 << (n_ent - 1).bit_length() - 1
    if ent_half < n_ent - ent_half:
        ent_half = n_ent
    rel_half = max(_BL, 1 << (n_rel - 1).bit_length() - 1)
    ent2 = _pack_table(entity_emb.T, ent_half)
    rel2 = _pack_table(relation_emb.T, rel_half)

    mesh = plsc.VectorSubcoreMesh(core_axis_name="c", subcore_axis_name="s")
    partials = pl.kernel(
        functools.partial(_tec_body, ent_half, rel_half),
        out_type=jax.ShapeDtypeStruct((_NW, 8, 128), jnp.float32),
        mesh=mesh,
        compiler_params=pltpu.CompilerParams(needs_layout_passes=False,
                                             use_tc_tiling_on_sc=True),
        scratch_types=[
            pltpu.VMEM((_BPW,), jnp.int32),
            pltpu.VMEM((_BPW,), jnp.int32),
            pltpu.VMEM((_BPW,), jnp.int32),
            pltpu.VMEM((_BPW,), jnp.int32),
            pltpu.VMEM((_BPW,), jnp.int32),
            pltpu.VMEM((5, _BPW), jnp.int32),
            pltpu.VMEM((2, _CHUNK, 2 * _DIM), jnp.float32),
            pltpu.VMEM((2, _CHUNK, 2 * _DIM), jnp.float32),
            pltpu.VMEM((2, _CHUNK, 2 * _DIM), jnp.float32),
            pltpu.VMEM((2, _CHUNK, 2 * _DIM), jnp.float32),
            pltpu.VMEM((2, _CHUNK, 2 * _DIM), jnp.float32),
            pltpu.VMEM((8, 128), jnp.float32),
            pltpu.SemaphoreType.DMA((2,)),
        ],
    )(heads, relations, tails, negative_heads, negative_tails, ent2, rel2)
    sums = jnp.sum(partials[:, 0:3, 0:16], axis=(0, 2))
    inv_b = 1.0 / _BATCH
    return sums[0] * inv_b, sums[1] * inv_b, sums[2] * inv_b


def kernel(heads, relations, tails, negative_heads, negative_tails,
           entity_emb, relation_emb):
    return _transe_sc(heads.astype(jnp.int32), relations.astype(jnp.int32),
                      tails.astype(jnp.int32),
                      negative_heads.astype(jnp.int32),
                      negative_tails.astype(jnp.int32),
                      entity_emb, relation_emb)


# speedup vs baseline: 1.1517x; 1.1517x over previous
import functools

import jax
import jax.numpy as jnp
import numpy as np
from jax import lax
from jax.experimental import pallas as pl
from jax.experimental.pallas import tpu as pltpu
from jax.experimental.pallas import tpu_sc as plsc

_BATCH = 16384
_DIM = 64
_GAMMA = 12.0
_NW = 32
_BPW = _BATCH // _NW
_CHUNK = 64
_NCHUNK = _BPW // _CHUNK
_GROUPS = _CHUNK // 16
_BL = 2048


def _pack_body(eye_ref, lo_ref, hi_ref, out_ref):
    x = jnp.concatenate([lo_ref[...], hi_ref[...]], axis=0)
    out_ref[...] = jax.lax.dot_general(
        x, eye_ref[...],
        dimension_numbers=(((0,), (0,)), ((), ())),
        preferred_element_type=jnp.float32,
        precision=jax.lax.Precision.DEFAULT)


def _pack_table(table_t, half_rows):
    n = table_t.shape[1]
    nlo = half_rows // _BL
    max_blk = (n + _BL - 1) // _BL - 1

    def lo_map(i):
        return (0, i)

    def hi_map(i):
        return (0, jnp.minimum(nlo + i, max_blk))

    eye = np.zeros((2 * _DIM, 2 * _DIM), np.float32)
    eye[:_DIM, :_DIM] = np.eye(_DIM, dtype=np.float32)
    eye[_DIM:, _DIM:] = np.eye(_DIM, dtype=np.float32)
    return pl.pallas_call(
        _pack_body,
        grid=(nlo,),
        in_specs=[
            pl.BlockSpec((2 * _DIM, 2 * _DIM), lambda i: (0, 0)),
            pl.BlockSpec((_DIM, _BL), lo_map),
            pl.BlockSpec((_DIM, _BL), hi_map),
        ],
        out_specs=pl.BlockSpec((_BL, 2 * _DIM), lambda i: (i, 0)),
        out_shape=jax.ShapeDtypeStruct((half_rows, 2 * _DIM), jnp.float32),
    )(jnp.asarray(eye), table_t, table_t)


def _tec_body(ent_half, rel_half,
              heads_h, rels_h, tails_h, nheads_h, ntails_h, ent_h, rel_h,
              out_h,
              h_idx, r_idx, t_idx, nh_idx, nt_idx, row_idx,
              h_rows, r_rows, t_rows, nh_rows, nt_rows,
              out_stage, sems):
    wid = lax.axis_index("s") * 2 + lax.axis_index("c")
    base = wid * _BPW

    pltpu.sync_copy(heads_h.at[pl.ds(base, _BPW)], h_idx)
    pltpu.sync_copy(rels_h.at[pl.ds(base, _BPW)], r_idx)
    pltpu.sync_copy(tails_h.at[pl.ds(base, _BPW)], t_idx)
    pltpu.sync_copy(nheads_h.at[pl.ds(base, _BPW)], nh_idx)
    pltpu.sync_copy(ntails_h.at[pl.ds(base, _BPW)], nt_idx)

    idx_bufs = (h_idx, r_idx, t_idx, nh_idx, nt_idx)
    halves = (ent_half, rel_half, ent_half, ent_half, ent_half)

    def shift_body(i, _):
        for b in range(5):
            v = idx_bufs[b][pl.ds(i * 16, 16)]
            hv = jnp.full((16,), halves[b], jnp.int32)
            row_idx[b, pl.ds(i * 16, 16)] = jnp.where(v < hv, v, v - hv)
        return 0

    lax.fori_loop(0, _BPW // 16, shift_body, 0, unroll=2)

    def start_chunk(c, buf_par):
        off = c * _CHUNK
        sem = sems.at[buf_par]
        srcs = (ent_h, rel_h, ent_h, ent_h, ent_h)
        dsts = (h_rows, r_rows, t_rows, nh_rows, nt_rows)
        return [
            pltpu.async_copy(
                srcs[b].at[row_idx.at[b, pl.ds(off, _CHUNK)]],
                dsts[b].at[buf_par], sem)
            for b in range(5)
        ]

    zero = jnp.zeros((16,), jnp.float32)
    v_loss, v_pos, v_neg = zero, zero, zero
    gamma = zero + _GAMMA
    iota = lax.iota(jnp.int32, 16)

    descs = [None, None]
    descs[0] = start_chunk(0, 0)

    for c in range(_NCHUNK):
        buf_par = c % 2
        if c + 1 < _NCHUNK:
            descs[(c + 1) % 2] = start_chunk(c + 1, (c + 1) % 2)
        for d in descs[buf_par]:
            d.wait()

        hb, rb, tb = h_rows.at[buf_par], r_rows.at[buf_par], t_rows.at[buf_par]
        nhb, ntb = nh_rows.at[buf_par], nt_rows.at[buf_par]

        for g in range(_GROUPS):
            goff = c * _CHUNK + g * 16
            rows = iota + g * 16
            cols = []
            for b in range(5):
                v = idx_bufs[b][pl.ds(goff, 16)]
                hv = jnp.full((16,), halves[b], jnp.int32)
                cols.append(jnp.where(v < hv, 0, _DIM))

            def dim_body(d, acc, hb=hb, rb=rb, tb=tb, nhb=nhb, ntb=ntb,
                         rows=rows, cols=cols):
                ap, an = acc
                h = plsc.load_gather(hb, [rows, cols[0] + d])
                r = plsc.load_gather(rb, [rows, cols[1] + d])
                t = plsc.load_gather(tb, [rows, cols[2] + d])
                nh = plsc.load_gather(nhb, [rows, cols[3] + d])
                nt = plsc.load_gather(ntb, [rows, cols[4] + d])
                ap = ap + jnp.abs(h + r - t)
                an = an + jnp.abs(nh + r - nt)
                return ap, an

            ap, an = lax.fori_loop(0, _DIM, dim_body, (zero, zero), unroll=2)
            v_loss = v_loss + jnp.maximum(gamma + ap - an, 0.0)
            v_pos = v_pos + ap
            v_neg = v_neg + an

    out_stage[0, pl.ds(0, 16)] = v_loss
    out_stage[1, pl.ds(0, 16)] = v_pos
    out_stage[2, pl.ds(0, 16)] = v_neg
    pltpu.sync_copy(out_stage, out_h.at[wid])


@jax.jit
def _transe_sc(heads, relations, tails, negative_heads, negative_tails,
               entity_emb, relation_emb):
    n_ent = entity_emb.shape[0]
    n_rel = relation_emb.shape[0]
    ent_half = 1 << (n_ent - 1).bit_length() - 1
    if ent_half < n_ent - ent_half:
        ent_half = n_ent
    rel_half = max(_BL, 1 << (n_rel - 1).bit_length() - 1)
    ent2 = _pack_table(entity_emb.T, ent_half)
    rel2 = _pack_table(relation_emb.T, rel_half)

    mesh = plsc.VectorSubcoreMesh(core_axis_name="c", subcore_axis_name="s")
    partials = pl.kernel(
        functools.partial(_tec_body, ent_half, rel_half),
        out_type=jax.ShapeDtypeStruct((_NW, 8, 128), jnp.float32),
        mesh=mesh,
        compiler_params=pltpu.CompilerParams(needs_layout_passes=False,
                                             use_tc_tiling_on_sc=True),
        scratch_types=[
            pltpu.VMEM((_BPW,), jnp.int32),
            pltpu.VMEM((_BPW,), jnp.int32),
            pltpu.VMEM((_BPW,), jnp.int32),
            pltpu.VMEM((_BPW,), jnp.int32),
            pltpu.VMEM((_BPW,), jnp.int32),
            pltpu.VMEM((5, _BPW), jnp.int32),
            pltpu.VMEM((2, _CHUNK, 2 * _DIM), jnp.float32),
            pltpu.VMEM((2, _CHUNK, 2 * _DIM), jnp.float32),
            pltpu.VMEM((2, _CHUNK, 2 * _DIM), jnp.float32),
            pltpu.VMEM((2, _CHUNK, 2 * _DIM), jnp.float32),
            pltpu.VMEM((2, _CHUNK, 2 * _DIM), jnp.float32),
            pltpu.VMEM((8, 128), jnp.float32),
            pltpu.SemaphoreType.DMA((2,)),
        ],
    )(heads, relations, tails, negative_heads, negative_tails, ent2, rel2)
    sums = jnp.sum(partials[:, 0:3, 0:16], axis=(0, 2))
    inv_b = 1.0 / _BATCH
    return sums[0] * inv_b, sums[1] * inv_b, sums[2] * inv_b


def kernel(heads, relations, tails, negative_heads, negative_tails,
           entity_emb, relation_emb):
    return _transe_sc(heads.astype(jnp.int32), relations.astype(jnp.int32),
                      tails.astype(jnp.int32),
                      negative_heads.astype(jnp.int32),
                      negative_tails.astype(jnp.int32),
                      entity_emb, relation_emb)


# speedup vs baseline: 1.4516x; 1.2604x over previous
import functools

import jax
import jax.numpy as jnp
import numpy as np
from jax import lax
from jax.experimental import pallas as pl
from jax.experimental.pallas import tpu as pltpu
from jax.experimental.pallas import tpu_sc as plsc

_BATCH = 16384
_DIM = 64
_GAMMA = 12.0
_NW = 32
_BPW = _BATCH // _NW
_CHUNK = 64
_NCHUNK = _BPW // _CHUNK
_GROUPS = _CHUNK // 16
_BL = 2048


def _pack_body(eye_ref, lo_a, hi_a, lo_b, hi_b, out_ref):
    xa = jnp.concatenate([lo_a[...], hi_a[...]], axis=0)
    xb = jnp.concatenate([lo_b[...], hi_b[...]], axis=0)
    dn = (((0,), (0,)), ((), ()))
    out_ref[0:_BL, :] = jax.lax.dot_general(
        xa, eye_ref[...], dimension_numbers=dn,
        preferred_element_type=jnp.float32,
        precision=jax.lax.Precision.DEFAULT)
    out_ref[_BL:2 * _BL, :] = jax.lax.dot_general(
        xb, eye_ref[...], dimension_numbers=dn,
        preferred_element_type=jnp.float32,
        precision=jax.lax.Precision.DEFAULT)


def _pack_table(table_t, half_rows):
    n = table_t.shape[1]
    nlo = half_rows // _BL
    max_blk = (n + _BL - 1) // _BL - 1

    def lo_a_map(i):
        return (0, 2 * i)

    def lo_b_map(i):
        return (0, 2 * i + 1)

    def hi_a_map(i):
        return (0, jnp.minimum(nlo + 2 * i, max_blk))

    def hi_b_map(i):
        return (0, jnp.minimum(nlo + 2 * i + 1, max_blk))

    eye = np.zeros((2 * _DIM, 2 * _DIM), np.float32)
    eye[:_DIM, :_DIM] = np.eye(_DIM, dtype=np.float32)
    eye[_DIM:, _DIM:] = np.eye(_DIM, dtype=np.float32)
    return pl.pallas_call(
        _pack_body,
        grid=(nlo // 2,),
        in_specs=[
            pl.BlockSpec((2 * _DIM, 2 * _DIM), lambda i: (0, 0)),
            pl.BlockSpec((_DIM, _BL), lo_a_map),
            pl.BlockSpec((_DIM, _BL), hi_a_map),
            pl.BlockSpec((_DIM, _BL), lo_b_map),
            pl.BlockSpec((_DIM, _BL), hi_b_map),
        ],
        out_specs=pl.BlockSpec((2 * _BL, 2 * _DIM), lambda i: (i, 0)),
        out_shape=jax.ShapeDtypeStruct((half_rows, 2 * _DIM), jnp.float32),
    )(jnp.asarray(eye), table_t, table_t, table_t, table_t)


def _tec_body(ent_half, rel_half,
              heads_h, rels_h, tails_h, nheads_h, ntails_h, ent_h, rel_h,
              out_h,
              h_idx, r_idx, t_idx, nh_idx, nt_idx, row_idx,
              h_rows, r_rows, t_rows, nh_rows, nt_rows,
              out_stage, sems):
    wid = lax.axis_index("s") * 2 + lax.axis_index("c")
    base = wid * _BPW

    pltpu.sync_copy(heads_h.at[pl.ds(base, _BPW)], h_idx)
    pltpu.sync_copy(rels_h.at[pl.ds(base, _BPW)], r_idx)
    pltpu.sync_copy(tails_h.at[pl.ds(base, _BPW)], t_idx)
    pltpu.sync_copy(nheads_h.at[pl.ds(base, _BPW)], nh_idx)
    pltpu.sync_copy(ntails_h.at[pl.ds(base, _BPW)], nt_idx)

    idx_bufs = (h_idx, r_idx, t_idx, nh_idx, nt_idx)
    halves = (ent_half, rel_half, ent_half, ent_half, ent_half)

    def shift_body(i, _):
        for b in range(5):
            v = idx_bufs[b][pl.ds(i * 16, 16)]
            hv = jnp.full((16,), halves[b], jnp.int32)
            row_idx[b, pl.ds(i * 16, 16)] = jnp.where(v < hv, v, v - hv)
        return 0

    lax.fori_loop(0, _BPW // 16, shift_body, 0, unroll=2)

    def start_chunk(c, buf_par):
        off = c * _CHUNK
        sem = sems.at[buf_par]
        srcs = (ent_h, rel_h, ent_h, ent_h, ent_h)
        dsts = (h_rows, r_rows, t_rows, nh_rows, nt_rows)
        return [
            pltpu.async_copy(
                srcs[b].at[row_idx.at[b, pl.ds(off, _CHUNK)]],
                dsts[b].at[buf_par], sem)
            for b in range(5)
        ]

    zero = jnp.zeros((16,), jnp.float32)
    v_loss, v_pos, v_neg = zero, zero, zero
    gamma = zero + _GAMMA
    iota = lax.iota(jnp.int32, 16)

    descs = [None, None]
    descs[0] = start_chunk(0, 0)

    for c in range(_NCHUNK):
        buf_par = c % 2
        if c + 1 < _NCHUNK:
            descs[(c + 1) % 2] = start_chunk(c + 1, (c + 1) % 2)
        for d in descs[buf_par]:
            d.wait()

        hb, rb, tb = h_rows.at[buf_par], r_rows.at[buf_par], t_rows.at[buf_par]
        nhb, ntb = nh_rows.at[buf_par], nt_rows.at[buf_par]

        for g in range(_GROUPS):
            goff = c * _CHUNK + g * 16
            rows = iota + g * 16
            cols = []
            for b in range(5):
                v = idx_bufs[b][pl.ds(goff, 16)]
                hv = jnp.full((16,), halves[b], jnp.int32)
                cols.append(jnp.where(v < hv, 0, _DIM))

            def dim_body(d, acc, hb=hb, rb=rb, tb=tb, nhb=nhb, ntb=ntb,
                         rows=rows, cols=cols):
                ap, an = acc
                h = plsc.load_gather(hb, [rows, cols[0] + d])
                r = plsc.load_gather(rb, [rows, cols[1] + d])
                t = plsc.load_gather(tb, [rows, cols[2] + d])
                nh = plsc.load_gather(nhb, [rows, cols[3] + d])
                nt = plsc.load_gather(ntb, [rows, cols[4] + d])
                ap = ap + jnp.abs(h + r - t)
                an = an + jnp.abs(nh + r - nt)
                return ap, an

            ap, an = lax.fori_loop(0, _DIM, dim_body, (zero, zero), unroll=2)
            v_loss = v_loss + jnp.maximum(gamma + ap - an, 0.0)
            v_pos = v_pos + ap
            v_neg = v_neg + an

    out_stage[0, pl.ds(0, 16)] = v_loss
    out_stage[1, pl.ds(0, 16)] = v_pos
    out_stage[2, pl.ds(0, 16)] = v_neg
    pltpu.sync_copy(out_stage, out_h.at[wid])


@jax.jit
def _transe_sc(heads, relations, tails, negative_heads, negative_tails,
               entity_emb, relation_emb):
    n_ent = entity_emb.shape[0]
    n_rel = relation_emb.shape[0]
    ent_half = 1 << (n_ent - 1).bit_length() - 1
    if ent_half < n_ent - ent_half:
        ent_half = n_ent
    rel_half = max(_BL, 1 << (n_rel - 1).bit_length() - 1)
    ent2 = _pack_table(entity_emb.T, ent_half)
    rel2 = _pack_table(relation_emb.T, rel_half)

    mesh = plsc.VectorSubcoreMesh(core_axis_name="c", subcore_axis_name="s")
    partials = pl.kernel(
        functools.partial(_tec_body, ent_half, rel_half),
        out_type=jax.ShapeDtypeStruct((_NW, 8, 128), jnp.float32),
        mesh=mesh,
        compiler_params=pltpu.CompilerParams(needs_layout_passes=False,
                                             use_tc_tiling_on_sc=True),
        scratch_types=[
            pltpu.VMEM((_BPW,), jnp.int32),
            pltpu.VMEM((_BPW,), jnp.int32),
            pltpu.VMEM((_BPW,), jnp.int32),
            pltpu.VMEM((_BPW,), jnp.int32),
            pltpu.VMEM((_BPW,), jnp.int32),
            pltpu.VMEM((5, _BPW), jnp.int32),
            pltpu.VMEM((2, _CHUNK, 2 * _DIM), jnp.float32),
            pltpu.VMEM((2, _CHUNK, 2 * _DIM), jnp.float32),
            pltpu.VMEM((2, _CHUNK, 2 * _DIM), jnp.float32),
            pltpu.VMEM((2, _CHUNK, 2 * _DIM), jnp.float32),
            pltpu.VMEM((2, _CHUNK, 2 * _DIM), jnp.float32),
            pltpu.VMEM((8, 128), jnp.float32),
            pltpu.SemaphoreType.DMA((2,)),
        ],
    )(heads, relations, tails, negative_heads, negative_tails, ent2, rel2)
    sums = jnp.sum(partials[:, 0:3, 0:16], axis=(0, 2))
    inv_b = 1.0 / _BATCH
    return sums[0] * inv_b, sums[1] * inv_b, sums[2] * inv_b


def kernel(heads, relations, tails, negative_heads, negative_tails,
           entity_emb, relation_emb):
    return _transe_sc(heads.astype(jnp.int32), relations.astype(jnp.int32),
                      tails.astype(jnp.int32),
                      negative_heads.astype(jnp.int32),
                      negative_tails.astype(jnp.int32),
                      entity_emb, relation_emb)


# speedup vs baseline: 1.5962x; 1.0996x over previous
import functools

import jax
import jax.numpy as jnp
import numpy as np
from jax import lax
from jax.experimental import pallas as pl
from jax.experimental.pallas import tpu as pltpu
from jax.experimental.pallas import tpu_sc as plsc

_BATCH = 16384
_DIM = 64
_GAMMA = 12.0
_NW = 32
_BPW = _BATCH // _NW
_CHUNK = 64
_NCHUNK = _BPW // _CHUNK
_GROUPS = _CHUNK // 16
_BL = 2048


def _pack_body(ways, eye_ref, *refs):
    dn = (((0,), (0,)), ((), ()))
    out_ref = refs[-1]
    for j in range(ways):
        lo, hi = refs[2 * j], refs[2 * j + 1]
        x = jnp.concatenate([lo[...], hi[...]], axis=0)
        out_ref[j * _BL:(j + 1) * _BL, :] = jax.lax.dot_general(
            x, eye_ref[...], dimension_numbers=dn,
            preferred_element_type=jnp.float32,
            precision=jax.lax.Precision.DEFAULT)


def _pack_table(table_t, half_rows):
    n = table_t.shape[1]
    nlo = half_rows // _BL
    max_blk = (n + _BL - 1) // _BL - 1

    ways = 4 if nlo % 4 == 0 else 1

    def mk_lo_map(j):
        def m(i):
            return (0, ways * i + j)
        return m

    def mk_hi_map(j):
        def m(i):
            return (0, jnp.minimum(nlo + ways * i + j, max_blk))
        return m

    eye = np.zeros((2 * _DIM, 2 * _DIM), np.float32)
    eye[:_DIM, :_DIM] = np.eye(_DIM, dtype=np.float32)
    eye[_DIM:, _DIM:] = np.eye(_DIM, dtype=np.float32)
    in_specs = [pl.BlockSpec((2 * _DIM, 2 * _DIM), lambda i: (0, 0))]
    for j in range(ways):
        in_specs.append(pl.BlockSpec((_DIM, _BL), mk_lo_map(j)))
        in_specs.append(pl.BlockSpec((_DIM, _BL), mk_hi_map(j)))
    return pl.pallas_call(
        functools.partial(_pack_body, ways),
        grid=(nlo // ways,),
        in_specs=in_specs,
        out_specs=pl.BlockSpec((ways * _BL, 2 * _DIM), lambda i: (i, 0)),
        out_shape=jax.ShapeDtypeStruct((half_rows, 2 * _DIM), jnp.float32),
    )(jnp.asarray(eye), *([table_t] * (2 * ways)))


def _tec_body(ent_half, rel_half,
              heads_h, rels_h, tails_h, nheads_h, ntails_h, ent_h, rel_h,
              out_h,
              h_idx, r_idx, t_idx, nh_idx, nt_idx, row_idx,
              h_rows, r_rows, t_rows, nh_rows, nt_rows,
              out_stage, sems):
    wid = lax.axis_index("s") * 2 + lax.axis_index("c")
    base = wid * _BPW

    pltpu.sync_copy(heads_h.at[pl.ds(base, _BPW)], h_idx)
    pltpu.sync_copy(rels_h.at[pl.ds(base, _BPW)], r_idx)
    pltpu.sync_copy(tails_h.at[pl.ds(base, _BPW)], t_idx)
    pltpu.sync_copy(nheads_h.at[pl.ds(base, _BPW)], nh_idx)
    pltpu.sync_copy(ntails_h.at[pl.ds(base, _BPW)], nt_idx)

    idx_bufs = (h_idx, r_idx, t_idx, nh_idx, nt_idx)
    halves = (ent_half, rel_half, ent_half, ent_half, ent_half)

    def shift_body(i, _):
        for b in range(5):
            v = idx_bufs[b][pl.ds(i * 16, 16)]
            hv = jnp.full((16,), halves[b], jnp.int32)
            row_idx[b, pl.ds(i * 16, 16)] = jnp.where(v < hv, v, v - hv)
        return 0

    lax.fori_loop(0, _BPW // 16, shift_body, 0, unroll=2)

    def start_chunk(c, buf_par):
        off = c * _CHUNK
        sem = sems.at[buf_par]
        srcs = (ent_h, rel_h, ent_h, ent_h, ent_h)
        dsts = (h_rows, r_rows, t_rows, nh_rows, nt_rows)
        return [
            pltpu.async_copy(
                srcs[b].at[row_idx.at[b, pl.ds(off, _CHUNK)]],
                dsts[b].at[buf_par], sem)
            for b in range(5)
        ]

    zero = jnp.zeros((16,), jnp.float32)
    v_loss, v_pos, v_neg = zero, zero, zero
    gamma = zero + _GAMMA
    iota = lax.iota(jnp.int32, 16)

    descs = [None, None]
    descs[0] = start_chunk(0, 0)

    for c in range(_NCHUNK):
        buf_par = c % 2
        if c + 1 < _NCHUNK:
            descs[(c + 1) % 2] = start_chunk(c + 1, (c + 1) % 2)
        for d in descs[buf_par]:
            d.wait()

        hb, rb, tb = h_rows.at[buf_par], r_rows.at[buf_par], t_rows.at[buf_par]
        nhb, ntb = nh_rows.at[buf_par], nt_rows.at[buf_par]

        for g in range(_GROUPS):
            goff = c * _CHUNK + g * 16
            rows = iota + g * 16
            cols = []
            for b in range(5):
                v = idx_bufs[b][pl.ds(goff, 16)]
                hv = jnp.full((16,), halves[b], jnp.int32)
                cols.append(jnp.where(v < hv, 0, _DIM))

            def dim_body(d, acc, hb=hb, rb=rb, tb=tb, nhb=nhb, ntb=ntb,
                         rows=rows, cols=cols):
                ap, an = acc
                h = plsc.load_gather(hb, [rows, cols[0] + d])
                r = plsc.load_gather(rb, [rows, cols[1] + d])
                t = plsc.load_gather(tb, [rows, cols[2] + d])
                nh = plsc.load_gather(nhb, [rows, cols[3] + d])
                nt = plsc.load_gather(ntb, [rows, cols[4] + d])
                ap = ap + jnp.abs(h + r - t)
                an = an + jnp.abs(nh + r - nt)
                return ap, an

            ap, an = lax.fori_loop(0, _DIM, dim_body, (zero, zero), unroll=2)
            v_loss = v_loss + jnp.maximum(gamma + ap - an, 0.0)
            v_pos = v_pos + ap
            v_neg = v_neg + an

    out_stage[0, pl.ds(0, 16)] = v_loss
    out_stage[1, pl.ds(0, 16)] = v_pos
    out_stage[2, pl.ds(0, 16)] = v_neg
    pltpu.sync_copy(out_stage, out_h.at[wid])


@jax.jit
def _transe_sc(heads, relations, tails, negative_heads, negative_tails,
               entity_emb, relation_emb):
    n_ent = entity_emb.shape[0]
    n_rel = relation_emb.shape[0]
    ent_half = 1 << (n_ent - 1).bit_length() - 1
    if ent_half < n_ent - ent_half:
        ent_half = n_ent
    rel_half = max(_BL, 1 << (n_rel - 1).bit_length() - 1)
    ent2 = _pack_table(entity_emb.T, ent_half)
    rel2 = _pack_table(relation_emb.T, rel_half)

    mesh = plsc.VectorSubcoreMesh(core_axis_name="c", subcore_axis_name="s")
    partials = pl.kernel(
        functools.partial(_tec_body, ent_half, rel_half),
        out_type=jax.ShapeDtypeStruct((_NW, 8, 128), jnp.float32),
        mesh=mesh,
        compiler_params=pltpu.CompilerParams(needs_layout_passes=False,
                                             use_tc_tiling_on_sc=True),
        scratch_types=[
            pltpu.VMEM((_BPW,), jnp.int32),
            pltpu.VMEM((_BPW,), jnp.int32),
            pltpu.VMEM((_BPW,), jnp.int32),
            pltpu.VMEM((_BPW,), jnp.int32),
            pltpu.VMEM((_BPW,), jnp.int32),
            pltpu.VMEM((5, _BPW), jnp.int32),
            pltpu.VMEM((2, _CHUNK, 2 * _DIM), jnp.float32),
            pltpu.VMEM((2, _CHUNK, 2 * _DIM), jnp.float32),
            pltpu.VMEM((2, _CHUNK, 2 * _DIM), jnp.float32),
            pltpu.VMEM((2, _CHUNK, 2 * _DIM), jnp.float32),
            pltpu.VMEM((2, _CHUNK, 2 * _DIM), jnp.float32),
            pltpu.VMEM((8, 128), jnp.float32),
            pltpu.SemaphoreType.DMA((2,)),
        ],
    )(heads, relations, tails, negative_heads, negative_tails, ent2, rel2)
    sums = jnp.sum(partials[:, 0:3, 0:16], axis=(0, 2))
    inv_b = 1.0 / _BATCH
    return sums[0] * inv_b, sums[1] * inv_b, sums[2] * inv_b


def kernel(heads, relations, tails, negative_heads, negative_tails,
           entity_emb, relation_emb):
    return _transe_sc(heads.astype(jnp.int32), relations.astype(jnp.int32),
                      tails.astype(jnp.int32),
                      negative_heads.astype(jnp.int32),
                      negative_tails.astype(jnp.int32),
                      entity_emb, relation_emb)


# speedup vs baseline: 1.6238x; 1.0173x over previous
import functools

import jax
import jax.numpy as jnp
import numpy as np
from jax import lax
from jax.experimental import pallas as pl
from jax.experimental.pallas import tpu as pltpu
from jax.experimental.pallas import tpu_sc as plsc

_BATCH = 16384
_DIM = 64
_GAMMA = 12.0
_NW = 32
_BPW = _BATCH // _NW
_CHUNK = 64
_NCHUNK = _BPW // _CHUNK
_GROUPS = _CHUNK // 16
_BL = 2048


def _pack_body(ways, eye_ref, *refs):
    dn = (((0,), (0,)), ((), ()))
    out_ref = refs[-1]
    for j in range(ways):
        lo, hi = refs[2 * j], refs[2 * j + 1]
        x = jnp.concatenate([lo[...], hi[...]], axis=0)
        out_ref[j * _BL:(j + 1) * _BL, :] = jax.lax.dot_general(
            x, eye_ref[...], dimension_numbers=dn,
            preferred_element_type=jnp.float32,
            precision=jax.lax.Precision.DEFAULT)


def _pack_table(table_t, half_rows):
    n = table_t.shape[1]
    nlo = half_rows // _BL
    max_blk = (n + _BL - 1) // _BL - 1

    ways = 8 if nlo % 8 == 0 else 1

    def mk_lo_map(j):
        def m(i):
            return (0, ways * i + j)
        return m

    def mk_hi_map(j):
        def m(i):
            return (0, jnp.minimum(nlo + ways * i + j, max_blk))
        return m

    eye = np.zeros((2 * _DIM, 2 * _DIM), np.float32)
    eye[:_DIM, :_DIM] = np.eye(_DIM, dtype=np.float32)
    eye[_DIM:, _DIM:] = np.eye(_DIM, dtype=np.float32)
    in_specs = [pl.BlockSpec((2 * _DIM, 2 * _DIM), lambda i: (0, 0))]
    for j in range(ways):
        in_specs.append(pl.BlockSpec((_DIM, _BL), mk_lo_map(j)))
        in_specs.append(pl.BlockSpec((_DIM, _BL), mk_hi_map(j)))
    return pl.pallas_call(
        functools.partial(_pack_body, ways),
        grid=(nlo // ways,),
        in_specs=in_specs,
        out_specs=pl.BlockSpec((ways * _BL, 2 * _DIM), lambda i: (i, 0)),
        out_shape=jax.ShapeDtypeStruct((half_rows, 2 * _DIM), jnp.float32),
    )(jnp.asarray(eye), *([table_t] * (2 * ways)))


def _tec_body(ent_half, rel_half,
              heads_h, rels_h, tails_h, nheads_h, ntails_h, ent_h, rel_h,
              out_h,
              h_idx, r_idx, t_idx, nh_idx, nt_idx, row_idx,
              h_rows, r_rows, t_rows, nh_rows, nt_rows,
              out_stage, sems):
    wid = lax.axis_index("s") * 2 + lax.axis_index("c")
    base = wid * _BPW

    pltpu.sync_copy(heads_h.at[pl.ds(base, _BPW)], h_idx)
    pltpu.sync_copy(rels_h.at[pl.ds(base, _BPW)], r_idx)
    pltpu.sync_copy(tails_h.at[pl.ds(base, _BPW)], t_idx)
    pltpu.sync_copy(nheads_h.at[pl.ds(base, _BPW)], nh_idx)
    pltpu.sync_copy(ntails_h.at[pl.ds(base, _BPW)], nt_idx)

    idx_bufs = (h_idx, r_idx, t_idx, nh_idx, nt_idx)
    halves = (ent_half, rel_half, ent_half, ent_half, ent_half)

    def shift_body(i, _):
        for b in range(5):
            v = idx_bufs[b][pl.ds(i * 16, 16)]
            hv = jnp.full((16,), halves[b], jnp.int32)
            row_idx[b, pl.ds(i * 16, 16)] = jnp.where(v < hv, v, v - hv)
        return 0

    lax.fori_loop(0, _BPW // 16, shift_body, 0, unroll=2)

    def start_chunk(c, buf_par):
        off = c * _CHUNK
        sem = sems.at[buf_par]
        srcs = (ent_h, rel_h, ent_h, ent_h, ent_h)
        dsts = (h_rows, r_rows, t_rows, nh_rows, nt_rows)
        return [
            pltpu.async_copy(
                srcs[b].at[row_idx.at[b, pl.ds(off, _CHUNK)]],
                dsts[b].at[buf_par], sem)
            for b in range(5)
        ]

    zero = jnp.zeros((16,), jnp.float32)
    v_loss, v_pos, v_neg = zero, zero, zero
    gamma = zero + _GAMMA
    iota = lax.iota(jnp.int32, 16)

    descs = [None, None]
    descs[0] = start_chunk(0, 0)

    for c in range(_NCHUNK):
        buf_par = c % 2
        if c + 1 < _NCHUNK:
            descs[(c + 1) % 2] = start_chunk(c + 1, (c + 1) % 2)
        for d in descs[buf_par]:
            d.wait()

        hb, rb, tb = h_rows.at[buf_par], r_rows.at[buf_par], t_rows.at[buf_par]
        nhb, ntb = nh_rows.at[buf_par], nt_rows.at[buf_par]

        for g in range(_GROUPS):
            goff = c * _CHUNK + g * 16
            rows = iota + g * 16
            cols = []
            for b in range(5):
                v = idx_bufs[b][pl.ds(goff, 16)]
                hv = jnp.full((16,), halves[b], jnp.int32)
                cols.append(jnp.where(v < hv, 0, _DIM))

            def dim_body(d, acc, hb=hb, rb=rb, tb=tb, nhb=nhb, ntb=ntb,
                         rows=rows, cols=cols):
                ap, an = acc
                h = plsc.load_gather(hb, [rows, cols[0] + d])
                r = plsc.load_gather(rb, [rows, cols[1] + d])
                t = plsc.load_gather(tb, [rows, cols[2] + d])
                nh = plsc.load_gather(nhb, [rows, cols[3] + d])
                nt = plsc.load_gather(ntb, [rows, cols[4] + d])
                ap = ap + jnp.abs(h + r - t)
                an = an + jnp.abs(nh + r - nt)
                return ap, an

            ap, an = lax.fori_loop(0, _DIM, dim_body, (zero, zero), unroll=2)
            v_loss = v_loss + jnp.maximum(gamma + ap - an, 0.0)
            v_pos = v_pos + ap
            v_neg = v_neg + an

    out_stage[0, pl.ds(0, 16)] = v_loss
    out_stage[1, pl.ds(0, 16)] = v_pos
    out_stage[2, pl.ds(0, 16)] = v_neg
    pltpu.sync_copy(out_stage, out_h.at[wid])


@jax.jit
def _transe_sc(heads, relations, tails, negative_heads, negative_tails,
               entity_emb, relation_emb):
    n_ent = entity_emb.shape[0]
    n_rel = relation_emb.shape[0]
    ent_half = 1 << (n_ent - 1).bit_length() - 1
    if ent_half < n_ent - ent_half:
        ent_half = n_ent
    rel_half = max(_BL, 1 << (n_rel - 1).bit_length() - 1)
    ent2 = _pack_table(entity_emb.T, ent_half)
    rel2 = _pack_table(relation_emb.T, rel_half)

    mesh = plsc.VectorSubcoreMesh(core_axis_name="c", subcore_axis_name="s")
    partials = pl.kernel(
        functools.partial(_tec_body, ent_half, rel_half),
        out_type=jax.ShapeDtypeStruct((_NW, 8, 128), jnp.float32),
        mesh=mesh,
        compiler_params=pltpu.CompilerParams(needs_layout_passes=False,
                                             use_tc_tiling_on_sc=True),
        scratch_types=[
            pltpu.VMEM((_BPW,), jnp.int32),
            pltpu.VMEM((_BPW,), jnp.int32),
            pltpu.VMEM((_BPW,), jnp.int32),
            pltpu.VMEM((_BPW,), jnp.int32),
            pltpu.VMEM((_BPW,), jnp.int32),
            pltpu.VMEM((5, _BPW), jnp.int32),
            pltpu.VMEM((2, _CHUNK, 2 * _DIM), jnp.float32),
            pltpu.VMEM((2, _CHUNK, 2 * _DIM), jnp.float32),
            pltpu.VMEM((2, _CHUNK, 2 * _DIM), jnp.float32),
            pltpu.VMEM((2, _CHUNK, 2 * _DIM), jnp.float32),
            pltpu.VMEM((2, _CHUNK, 2 * _DIM), jnp.float32),
            pltpu.VMEM((8, 128), jnp.float32),
            pltpu.SemaphoreType.DMA((2,)),
        ],
    )(heads, relations, tails, negative_heads, negative_tails, ent2, rel2)
    sums = jnp.sum(partials[:, 0:3, 0:16], axis=(0, 2))
    inv_b = 1.0 / _BATCH
    return sums[0] * inv_b, sums[1] * inv_b, sums[2] * inv_b


def kernel(heads, relations, tails, negative_heads, negative_tails,
           entity_emb, relation_emb):
    return _transe_sc(heads.astype(jnp.int32), relations.astype(jnp.int32),
                      tails.astype(jnp.int32),
                      negative_heads.astype(jnp.int32),
                      negative_tails.astype(jnp.int32),
                      entity_emb, relation_emb)
